# Initial kernel scaffold; baseline (speedup 1.0000x reference)
import jax
import jax.numpy as jnp
from jax.experimental import pallas as pl


def kernel(input, table):
    raise NotImplementedError("write your pallas kernel here")


# speedup vs baseline: 6.5126x; 6.5126x over previous
import functools

import jax
import jax.numpy as jnp
from jax import lax
from jax.experimental import pallas as pl
from jax.experimental.pallas import tpu as pltpu
from jax.experimental.pallas import tpu_sc as plsc

_CHUNK = 128
_NBUF = 2


@functools.lru_cache(maxsize=None)
def _make_gather(total, V, D, NC, NS):
    NW = NC * NS
    assert total % (NW * _CHUNK) == 0
    nstep = total // (NW * _CHUNK)
    b_per_w = nstep * _CHUNK
    mesh = plsc.VectorSubcoreMesh(core_axis_name="c", subcore_axis_name="s")

    @functools.partial(
        pl.kernel,
        out_type=jax.ShapeDtypeStruct((total, D), jnp.float32),
        mesh=mesh,
        scratch_types=[
            pltpu.VMEM((nstep, _CHUNK), jnp.int32),
            *[pltpu.VMEM((_CHUNK, D), jnp.float32) for _ in range(_NBUF)],
            *[pltpu.SemaphoreType.DMA for _ in range(_NBUF)],
        ],
    )
    def body(table_hbm, idx_hbm, out_hbm, idx_v, *rest):
        rows = rest[:_NBUF]
        sems = rest[_NBUF:]
        wid = lax.axis_index("s") * NC + lax.axis_index("c")
        base = wid * b_per_w
        pltpu.sync_copy(idx_hbm.at[wid], idx_v)

        for b in range(_NBUF):
            pltpu.async_copy(table_hbm.at[idx_v.at[b]], rows[b], sems[b])

        @pl.loop(0, nstep, step=_NBUF)
        def _(j0):
            for b in range(_NBUF):
                j = j0 + b
                pltpu.make_async_copy(
                    table_hbm.at[idx_v.at[b]], rows[b], sems[b]
                ).wait()
                pltpu.sync_copy(
                    rows[b], out_hbm.at[pl.ds(base + j * _CHUNK, _CHUNK)]
                )

                @pl.when(j + _NBUF < nstep)
                def _():
                    pltpu.async_copy(
                        table_hbm.at[idx_v.at[j + _NBUF]], rows[b], sems[b]
                    )

    return body


def kernel(input, table):
    B, H = input.shape
    V, D = table.shape
    info = plsc.get_sparse_core_info()
    NC, NS = info.num_cores, info.num_subcores
    total = B * H
    idx = input.astype(jnp.int32).reshape(NC * NS, total // (NC * NS * _CHUNK), _CHUNK)
    out = _make_gather(total, V, D, NC, NS)(table, idx)
    return out.reshape(B, H, D)


# speedup vs baseline: 15.6288x; 2.3998x over previous
import functools

import jax
import jax.numpy as jnp
from jax import lax
from jax.experimental import pallas as pl
from jax.experimental.pallas import tpu as pltpu
from jax.experimental.pallas import tpu_sc as plsc

_CHUNK = 128
_NBUF = 2


@functools.lru_cache(maxsize=None)
def _make_gather(total, V, D, NC, NS):
    NW = NC * NS
    assert total % (NW * _CHUNK) == 0
    nstep = total // (NW * _CHUNK)
    b_per_w = nstep * _CHUNK
    mesh = plsc.VectorSubcoreMesh(core_axis_name="c", subcore_axis_name="s")

    @functools.partial(
        pl.kernel,
        out_type=jax.ShapeDtypeStruct((total, D), jnp.float32),
        mesh=mesh,
        scratch_types=[
            pltpu.VMEM((nstep, _CHUNK), jnp.int32),
            pltpu.VMEM_SHARED((V, D), jnp.float32),
            *[pltpu.VMEM((_CHUNK, D), jnp.float32) for _ in range(_NBUF)],
            *[pltpu.SemaphoreType.DMA for _ in range(_NBUF)],
        ],
    )
    def body(table_hbm, idx_hbm, out_hbm, idx_v, table_s, *rest):
        rows = rest[:_NBUF]
        sems = rest[_NBUF:]
        sid = lax.axis_index("s")
        wid = sid * NC + lax.axis_index("c")
        base = wid * b_per_w
        @pl.when(sid == 0)
        def _():
            pltpu.sync_copy(table_hbm, table_s)

        pltpu.sync_copy(idx_hbm.at[wid], idx_v)
        plsc.subcore_barrier()

        for b in range(_NBUF):
            pltpu.async_copy(table_s.at[idx_v.at[b]], rows[b], sems[b])

        @pl.loop(0, nstep, step=_NBUF)
        def _(j0):
            for b in range(_NBUF):
                j = j0 + b
                pltpu.make_async_copy(
                    table_s.at[idx_v.at[b]], rows[b], sems[b]
                ).wait()
                pltpu.sync_copy(
                    rows[b], out_hbm.at[pl.ds(base + j * _CHUNK, _CHUNK)]
                )

                @pl.when(j + _NBUF < nstep)
                def _():
                    pltpu.async_copy(
                        table_s.at[idx_v.at[j + _NBUF]], rows[b], sems[b]
                    )

    return body


def kernel(input, table):
    B, H = input.shape
    V, D = table.shape
    info = plsc.get_sparse_core_info()
    NC, NS = info.num_cores, info.num_subcores
    total = B * H
    idx = input.astype(jnp.int32).reshape(NC * NS, total // (NC * NS * _CHUNK), _CHUNK)
    out = _make_gather(total, V, D, NC, NS)(table, idx)
    return out.reshape(B, H, D)


# speedup vs baseline: 15.9444x; 1.0202x over previous
import functools

import jax
import jax.numpy as jnp
from jax import lax
from jax.experimental import pallas as pl
from jax.experimental.pallas import tpu as pltpu
from jax.experimental.pallas import tpu_sc as plsc

_CHUNK = 128
_NBUF = 4
_K = 2


@functools.lru_cache(maxsize=None)
def _make_gather(total, V, D, NC, NS):
    NW = NC * NS
    assert total % (NW * _CHUNK) == 0
    nstep = total // (NW * _CHUNK)
    b_per_w = nstep * _CHUNK
    mesh = plsc.VectorSubcoreMesh(core_axis_name="c", subcore_axis_name="s")

    @functools.partial(
        pl.kernel,
        out_type=jax.ShapeDtypeStruct((total, D), jnp.float32),
        mesh=mesh,
        scratch_types=[
            pltpu.VMEM((nstep, _CHUNK), jnp.int32),
            pltpu.VMEM_SHARED((V, D), jnp.float32),
            *[pltpu.VMEM((_CHUNK, D), jnp.float32) for _ in range(_NBUF)],
            *[pltpu.SemaphoreType.DMA for _ in range(2 * _NBUF)],
        ],
    )
    def body(table_hbm, idx_hbm, out_hbm, idx_v, table_s, *rest):
        rows = rest[:_NBUF]
        gsems = rest[_NBUF : 2 * _NBUF]
        osems = rest[2 * _NBUF :]
        sid = lax.axis_index("s")
        wid = sid * NC + lax.axis_index("c")
        base = wid * b_per_w
        @pl.when(sid == 0)
        def _():
            pltpu.sync_copy(table_hbm, table_s)

        pltpu.sync_copy(idx_hbm.at[wid], idx_v)
        plsc.subcore_barrier()

        for m in range(_K):
            pltpu.async_copy(table_s.at[idx_v.at[m]], rows[m], gsems[m])

        @pl.loop(0, nstep, step=_NBUF)
        def _(j0):
            for b in range(_NBUF):
                j = j0 + b
                m = j + _K
                bm = (b + _K) % _NBUF

                @pl.when(m < nstep)
                def _():
                    @pl.when(j >= _NBUF - _K)
                    def _():
                        pltpu.make_async_copy(
                            rows[bm],
                            out_hbm.at[pl.ds(base, _CHUNK)],
                            osems[bm],
                        ).wait()

                    pltpu.async_copy(table_s.at[idx_v.at[m]], rows[bm], gsems[bm])

                pltpu.make_async_copy(
                    table_s.at[idx_v.at[b]], rows[b], gsems[b]
                ).wait()
                pltpu.async_copy(
                    rows[b], out_hbm.at[pl.ds(base + j * _CHUNK, _CHUNK)], osems[b]
                )

        for b in range(_NBUF):
            pltpu.make_async_copy(
                rows[b], out_hbm.at[pl.ds(base, _CHUNK)], osems[b]
            ).wait()

    return body


def kernel(input, table):
    B, H = input.shape
    V, D = table.shape
    info = plsc.get_sparse_core_info()
    NC, NS = info.num_cores, info.num_subcores
    total = B * H
    idx = input.astype(jnp.int32).reshape(NC * NS, total // (NC * NS * _CHUNK), _CHUNK)
    out = _make_gather(total, V, D, NC, NS)(table, idx)
    return out.reshape(B, H, D)
